# Initial kernel scaffold; baseline (speedup 1.0000x reference)
import jax
import jax.numpy as jnp
from jax.experimental import pallas as pl


def kernel(x, W1, att_src1, att_dst1, b1, W2, att_src2, att_dst2, b2, Wr1, br1, Wr2, br2, Wfc, bfc, Wh1, bh1, Wh2, bh2):
    raise NotImplementedError("write your pallas kernel here")


# speedup vs baseline: 2.9718x; 2.9718x over previous
import functools
import jax
import jax.numpy as jnp
from jax.experimental import pallas as pl
from jax.experimental.pallas import tpu as pltpu

KNN = 5
N = 10000
HEADS = 4
HID = 768


def _knn_body(xr_ref, xc_ref, o_ref, sim_ref, *, n_valid, R, C, NT, k):
    i = pl.program_id(0)
    j = pl.program_id(1)
    xr = xr_ref[...]
    xc = xc_ref[...]
    xr = xr * jax.lax.rsqrt(jnp.maximum((xr * xr).sum(1, keepdims=True), 1e-30))
    xc = xc * jax.lax.rsqrt(jnp.maximum((xc * xc).sum(1, keepdims=True), 1e-30))
    s = jax.lax.dot_general(xr, xc, (((1,), (1,)), ((), ())),
                            preferred_element_type=jnp.float32)
    row_g = i * R + jax.lax.broadcasted_iota(jnp.int32, (R, C), 0)
    col_g = j * C + jax.lax.broadcasted_iota(jnp.int32, (R, C), 1)
    s = jnp.where((col_g == row_g) | (col_g >= n_valid), -1e30, s)
    sim_ref[:, pl.ds(j * C, C)] = s

    @pl.when(j == pl.num_programs(1) - 1)
    def _():
        val = sim_ref[...]
        cols = jax.lax.broadcasted_iota(jnp.int32, (R, NT), 1)
        picks = []
        for _p in range(k):
            m = jnp.max(val, axis=1, keepdims=True)
            amin = jnp.min(jnp.where(val >= m, cols, NT), axis=1, keepdims=True)
            picks.append(amin)
            val = jnp.where(cols == amin, -jnp.float32(jnp.inf), val)
        picks += [jnp.zeros((R, 1), jnp.int32)] * (8 - k)
        o_ref[...] = jnp.concatenate(picks, axis=1)


def _knn_topk(xp, n_valid, k=KNN, R=64, C=512):
    NP, D = xp.shape
    grid = (NP // R, NP // C)
    out = pl.pallas_call(
        functools.partial(_knn_body, n_valid=n_valid, R=R, C=C, NT=NP, k=k),
        grid=grid,
        in_specs=[
            pl.BlockSpec((R, D), lambda i, j: (i, 0)),
            pl.BlockSpec((C, D), lambda i, j: (j, 0)),
        ],
        out_specs=pl.BlockSpec((R, 8), lambda i, j: (i, 0)),
        out_shape=jax.ShapeDtypeStruct((NP, 8), jnp.int32),
        scratch_shapes=[pltpu.VMEM((R, NP), jnp.float32)],
        compiler_params=pltpu.CompilerParams(
            dimension_semantics=("parallel", "arbitrary")),
    )(xp, xp)
    return out[:n_valid, :k]


def _gat_jnp(x, src, dst, W, a_src, a_dst, b, heads, out_dim):
    n = x.shape[0]
    loop = jnp.arange(n)
    s = jnp.concatenate([src, loop])
    d = jnp.concatenate([dst, loop])
    h = (x @ W).reshape(n, heads, out_dim)
    al_s = (h * a_src[None, :, :]).sum(-1)
    al_d = (h * a_dst[None, :, :]).sum(-1)
    alpha = al_s[s] + al_d[d]
    alpha = jax.nn.leaky_relu(alpha, negative_slope=0.2)
    amax = jax.ops.segment_max(alpha, d, num_segments=n)
    alpha = jnp.exp(alpha - amax[d])
    denom = jax.ops.segment_sum(alpha, d, num_segments=n)
    alpha = alpha / (denom[d] + 1e-16)
    msg = h[s] * alpha[:, :, None]
    out = jax.ops.segment_sum(msg, d, num_segments=n)
    return out.reshape(n, heads * out_dim) + b


def kernel(x, W1, att_src1, att_dst1, b1, W2, att_src2, att_dst2, b2,
           Wr1, br1, Wr2, br2, Wfc, bfc, Wh1, bh1, Wh2, bh2):
    n = x.shape[0]
    NP = 10240
    xp = jnp.pad(x, ((0, NP - n), (0, 0)))
    idx = _knn_topk(xp, n)
    src = jnp.repeat(jnp.arange(n), KNN)
    dst = idx.reshape(-1)

    x_res = x @ Wr1.T + br1
    h = jax.nn.relu(_gat_jnp(x, src, dst, W1, att_src1, att_dst1, b1,
                             HEADS, HID)) + x_res
    x_res2 = h @ Wr2.T + br2
    h2 = jax.nn.relu(_gat_jnp(h, src, dst, W2, att_src2, att_dst2, b2,
                              1, HID)) + x_res2
    feat = h2
    fc1 = jax.nn.relu(feat @ Wh1.T + bh1)
    feat_c = fc1 @ Wh2.T + bh2
    logits = feat @ Wfc.T + bfc
    norm = jnp.clip(jnp.linalg.norm(feat_c, axis=1, keepdims=True), 1e-12, None)
    return (logits, feat_c / norm)


# speedup vs baseline: 3.5991x; 1.2111x over previous
import functools
import jax
import jax.numpy as jnp
from jax.experimental import pallas as pl
from jax.experimental.pallas import tpu as pltpu

KNN = 5
N = 10000
HEADS = 4
HID = 768


def _knn_body(xr_ref, xc_ref, o_ref, sim_ref, *, n_valid, R, C, NT, k):
    i = pl.program_id(0)
    j = pl.program_id(1)
    xr = xr_ref[...]
    xc = xc_ref[...]
    xr = xr * jax.lax.rsqrt(jnp.maximum((xr * xr).sum(1, keepdims=True), 1e-30))
    xc = xc * jax.lax.rsqrt(jnp.maximum((xc * xc).sum(1, keepdims=True), 1e-30))
    s = jax.lax.dot_general(xr, xc, (((1,), (1,)), ((), ())),
                            preferred_element_type=jnp.float32)
    row_g = i * R + jax.lax.broadcasted_iota(jnp.int32, (R, C), 0)
    col_g = j * C + jax.lax.broadcasted_iota(jnp.int32, (R, C), 1)
    s = jnp.where((col_g == row_g) | (col_g >= n_valid), -1e30, s)
    sim_ref[:, pl.ds(j * C, C)] = s

    @pl.when(j == pl.num_programs(1) - 1)
    def _():
        val = sim_ref[...]
        cols = jax.lax.broadcasted_iota(jnp.int32, (R, NT), 1)
        picks = []
        for _p in range(k):
            m = jnp.max(val, axis=1, keepdims=True)
            amin = jnp.min(jnp.where(val >= m, cols, NT), axis=1, keepdims=True)
            picks.append(amin)
            val = jnp.where(cols == amin, -jnp.float32(jnp.inf), val)
        picks += [jnp.zeros((R, 1), jnp.int32)] * (8 - k)
        o_ref[...] = jnp.concatenate(picks, axis=1)


def _knn_topk(xp, n_valid, k=KNN, R=256, C=2048):
    NP, D = xp.shape
    grid = (NP // R, NP // C)
    out = pl.pallas_call(
        functools.partial(_knn_body, n_valid=n_valid, R=R, C=C, NT=NP, k=k),
        grid=grid,
        in_specs=[
            pl.BlockSpec((R, D), lambda i, j: (i, 0)),
            pl.BlockSpec((C, D), lambda i, j: (j, 0)),
        ],
        out_specs=pl.BlockSpec((R, 8), lambda i, j: (i, 0)),
        out_shape=jax.ShapeDtypeStruct((NP, 8), jnp.int32),
        scratch_shapes=[pltpu.VMEM((R, NP), jnp.float32)],
        compiler_params=pltpu.CompilerParams(
            dimension_semantics=("parallel", "arbitrary")),
    )(xp, xp)
    return out[:n_valid, :k]


def _gat_jnp(x, src, dst, W, a_src, a_dst, b, heads, out_dim):
    n = x.shape[0]
    loop = jnp.arange(n)
    s = jnp.concatenate([src, loop])
    d = jnp.concatenate([dst, loop])
    h = (x @ W).reshape(n, heads, out_dim)
    al_s = (h * a_src[None, :, :]).sum(-1)
    al_d = (h * a_dst[None, :, :]).sum(-1)
    alpha = al_s[s] + al_d[d]
    alpha = jax.nn.leaky_relu(alpha, negative_slope=0.2)
    amax = jax.ops.segment_max(alpha, d, num_segments=n)
    alpha = jnp.exp(alpha - amax[d])
    denom = jax.ops.segment_sum(alpha, d, num_segments=n)
    alpha = alpha / (denom[d] + 1e-16)
    msg = h[s] * alpha[:, :, None]
    out = jax.ops.segment_sum(msg, d, num_segments=n)
    return out.reshape(n, heads * out_dim) + b


def kernel(x, W1, att_src1, att_dst1, b1, W2, att_src2, att_dst2, b2,
           Wr1, br1, Wr2, br2, Wfc, bfc, Wh1, bh1, Wh2, bh2):
    n = x.shape[0]
    NP = 10240
    xp = jnp.pad(x, ((0, NP - n), (0, 0)))
    idx = _knn_topk(xp, n)
    src = jnp.repeat(jnp.arange(n), KNN)
    dst = idx.reshape(-1)

    x_res = x @ Wr1.T + br1
    h = jax.nn.relu(_gat_jnp(x, src, dst, W1, att_src1, att_dst1, b1,
                             HEADS, HID)) + x_res
    x_res2 = h @ Wr2.T + br2
    h2 = jax.nn.relu(_gat_jnp(h, src, dst, W2, att_src2, att_dst2, b2,
                              1, HID)) + x_res2
    feat = h2
    fc1 = jax.nn.relu(feat @ Wh1.T + bh1)
    feat_c = fc1 @ Wh2.T + bh2
    logits = feat @ Wfc.T + bfc
    norm = jnp.clip(jnp.linalg.norm(feat_c, axis=1, keepdims=True), 1e-12, None)
    return (logits, feat_c / norm)


# speedup vs baseline: 7.9652x; 2.2131x over previous
import functools
import jax
import jax.numpy as jnp
from jax.experimental import pallas as pl
from jax.experimental.pallas import tpu as pltpu

KNN = 5
HEADS = 4
HID = 768
NP = 10240


def _knn_body(xr_ref, xc_ref, o_ref, sim_ref, *, n_valid, R, C, NT, k):
    i = pl.program_id(0)
    j = pl.program_id(1)
    xr = xr_ref[...]
    xc = xc_ref[...]
    xr = xr * jax.lax.rsqrt(jnp.maximum((xr * xr).sum(1, keepdims=True), 1e-30))
    xc = xc * jax.lax.rsqrt(jnp.maximum((xc * xc).sum(1, keepdims=True), 1e-30))
    s = jax.lax.dot_general(xr, xc, (((1,), (1,)), ((), ())),
                            preferred_element_type=jnp.float32)
    row_g = i * R + jax.lax.broadcasted_iota(jnp.int32, (R, C), 0)
    col_g = j * C + jax.lax.broadcasted_iota(jnp.int32, (R, C), 1)
    s = jnp.where((col_g == row_g) | (col_g >= n_valid), -1e30, s)
    sim_ref[:, pl.ds(j * C, C)] = s

    @pl.when(j == pl.num_programs(1) - 1)
    def _():
        val = sim_ref[...]
        cols = jax.lax.broadcasted_iota(jnp.int32, (R, NT), 1)
        picks = []
        for _p in range(k):
            m = jnp.max(val, axis=1, keepdims=True)
            amin = jnp.min(jnp.where(val >= m, cols, NT), axis=1, keepdims=True)
            picks.append(amin)
            val = jnp.where(cols == amin, -jnp.float32(jnp.inf), val)
        picks += [jnp.zeros((R, 1), jnp.int32)] * (8 - k)
        o_ref[...] = jnp.concatenate(picks, axis=1)


def _knn_topk(xp, n_valid, k=KNN, R=256, C=2048):
    n_pad, D = xp.shape
    grid = (n_pad // R, n_pad // C)
    out = pl.pallas_call(
        functools.partial(_knn_body, n_valid=n_valid, R=R, C=C, NT=n_pad, k=k),
        grid=grid,
        in_specs=[
            pl.BlockSpec((R, D), lambda i, j: (i, 0)),
            pl.BlockSpec((C, D), lambda i, j: (j, 0)),
        ],
        out_specs=pl.BlockSpec((R, 8), lambda i, j: (i, 0)),
        out_shape=jax.ShapeDtypeStruct((n_pad, 8), jnp.int32),
        scratch_shapes=[pltpu.VMEM((R, n_pad), jnp.float32)],
        compiler_params=pltpu.CompilerParams(
            dimension_semantics=("parallel", "arbitrary")),
    )(xp, xp)
    return out


def _proj1_body(x_ref, W1_ref, Wr1_ref, as_ref, ad_ref, br1_ref,
                h1_ref, als_ref, ald_ref, xres_ref):
    x = x_ref[...]
    h1 = jax.lax.dot_general(x, W1_ref[...], (((1,), (0,)), ((), ())),
                             preferred_element_type=jnp.float32)
    h1_ref[...] = h1
    xres_ref[...] = jax.lax.dot_general(
        x, Wr1_ref[...], (((1,), (1,)), ((), ())),
        preferred_element_type=jnp.float32) + br1_ref[...]
    als, ald = [], []
    for h in range(HEADS):
        sl = h1[:, h * HID:(h + 1) * HID]
        als.append((sl * as_ref[h, :][None, :]).sum(1, keepdims=True))
        ald.append((sl * ad_ref[h, :][None, :]).sum(1, keepdims=True))
    zpad = [jnp.zeros_like(als[0])] * (8 - HEADS)
    als_ref[...] = jnp.concatenate(als + zpad, axis=1)
    ald_ref[...] = jnp.concatenate(ald + zpad, axis=1)


def _proj1(xp, W1, Wr1, asrc, adst, br1, R=256):
    D = xp.shape[1]
    DH = HEADS * HID
    grid = (NP // R,)
    return pl.pallas_call(
        _proj1_body,
        grid=grid,
        in_specs=[
            pl.BlockSpec((R, D), lambda i: (i, 0)),
            pl.BlockSpec((D, DH), lambda i: (0, 0)),
            pl.BlockSpec((DH, D), lambda i: (0, 0)),
            pl.BlockSpec((8, HID), lambda i: (0, 0)),
            pl.BlockSpec((8, HID), lambda i: (0, 0)),
            pl.BlockSpec((1, DH), lambda i: (0, 0)),
        ],
        out_specs=[
            pl.BlockSpec((R, DH), lambda i: (i, 0)),
            pl.BlockSpec((R, 8), lambda i: (i, 0)),
            pl.BlockSpec((R, 8), lambda i: (i, 0)),
            pl.BlockSpec((R, DH), lambda i: (i, 0)),
        ],
        out_shape=[
            jax.ShapeDtypeStruct((NP, DH), jnp.float32),
            jax.ShapeDtypeStruct((NP, 8), jnp.float32),
            jax.ShapeDtypeStruct((NP, 8), jnp.float32),
            jax.ShapeDtypeStruct((NP, DH), jnp.float32),
        ],
        compiler_params=pltpu.CompilerParams(
            dimension_semantics=("arbitrary",)),
    )(xp, W1, Wr1, asrc, adst, br1)


def _mid_body(h1_ref, nA_ref, nD_ref, als_ref, ald_ref, b1_ref, xres_ref,
              W2_ref, Wr2_ref, as2_ref, ad2_ref, br2_ref,
              h2_ref, als2_ref, ald2_ref, xres2_ref):
    als = als_ref[...]
    ald = ald_ref[...]
    a = als + ald
    aself = jnp.where(a > 0, a, 0.2 * a)
    es = jnp.exp(aself)
    denom = nD_ref[:, :8] + es + 1e-16
    h1 = h1_ref[...]
    nA = nA_ref[...]
    outs = []
    for h in range(HEADS):
        sl = slice(h * HID, (h + 1) * HID)
        agg = (nA[:, sl] + es[:, h:h + 1] * h1[:, sl]) / denom[:, h:h + 1]
        outs.append(agg)
    out1 = jnp.concatenate(outs, axis=1) + b1_ref[...]
    out1 = jnp.maximum(out1, 0.0) + xres_ref[...]
    h2 = jax.lax.dot_general(out1, W2_ref[...], (((1,), (0,)), ((), ())),
                             preferred_element_type=jnp.float32)
    h2_ref[...] = h2
    xres2_ref[...] = jax.lax.dot_general(
        out1, Wr2_ref[...], (((1,), (1,)), ((), ())),
        preferred_element_type=jnp.float32) + br2_ref[...]
    als2 = (h2 * as2_ref[0, :][None, :]).sum(1, keepdims=True)
    ald2 = (h2 * ad2_ref[0, :][None, :]).sum(1, keepdims=True)
    zpad = jnp.zeros((h2.shape[0], 7), jnp.float32)
    als2_ref[...] = jnp.concatenate([als2, zpad], axis=1)
    ald2_ref[...] = jnp.concatenate([ald2, zpad], axis=1)


def _mid(h1, nA1, nD1, als1, ald1, b1, xres1, W2, Wr2, as2, ad2, br2, R=256):
    DH = HEADS * HID
    grid = (NP // R,)
    return pl.pallas_call(
        _mid_body,
        grid=grid,
        in_specs=[
            pl.BlockSpec((R, DH), lambda i: (i, 0)),
            pl.BlockSpec((R, DH), lambda i: (i, 0)),
            pl.BlockSpec((R, 16), lambda i: (i, 0)),
            pl.BlockSpec((R, 8), lambda i: (i, 0)),
            pl.BlockSpec((R, 8), lambda i: (i, 0)),
            pl.BlockSpec((1, DH), lambda i: (0, 0)),
            pl.BlockSpec((R, DH), lambda i: (i, 0)),
            pl.BlockSpec((DH, HID), lambda i: (0, 0)),
            pl.BlockSpec((HID, DH), lambda i: (0, 0)),
            pl.BlockSpec((8, HID), lambda i: (0, 0)),
            pl.BlockSpec((8, HID), lambda i: (0, 0)),
            pl.BlockSpec((1, HID), lambda i: (0, 0)),
        ],
        out_specs=[
            pl.BlockSpec((R, HID), lambda i: (i, 0)),
            pl.BlockSpec((R, 8), lambda i: (i, 0)),
            pl.BlockSpec((R, 8), lambda i: (i, 0)),
            pl.BlockSpec((R, HID), lambda i: (i, 0)),
        ],
        out_shape=[
            jax.ShapeDtypeStruct((NP, HID), jnp.float32),
            jax.ShapeDtypeStruct((NP, 8), jnp.float32),
            jax.ShapeDtypeStruct((NP, 8), jnp.float32),
            jax.ShapeDtypeStruct((NP, HID), jnp.float32),
        ],
        compiler_params=pltpu.CompilerParams(
            dimension_semantics=("arbitrary",)),
    )(h1, nA1, nD1, als1, ald1, b1, xres1, W2, Wr2, as2, ad2, br2)


def _final_body(h2_ref, nA_ref, nD_ref, als_ref, ald_ref, b2_ref, xres_ref,
                Wh1_ref, bh1_ref, Wh2_ref, bh2_ref, Wfc_ref, bfc_ref,
                lg_ref, fn_ref):
    a = als_ref[:, 0:1] + ald_ref[:, 0:1]
    aself = jnp.where(a > 0, a, 0.2 * a)
    es = jnp.exp(aself)
    denom = nD_ref[:, 0:1] + es + 1e-16
    h2raw = h2_ref[...]
    agg = (nA_ref[...] + es * h2raw) / denom + b2_ref[...]
    feat = jnp.maximum(agg, 0.0) + xres_ref[...]
    fc1 = jax.lax.dot_general(feat, Wh1_ref[...], (((1,), (1,)), ((), ())),
                              preferred_element_type=jnp.float32) + bh1_ref[...]
    fc1 = jnp.maximum(fc1, 0.0)
    feat_c = jax.lax.dot_general(fc1, Wh2_ref[...], (((1,), (1,)), ((), ())),
                                 preferred_element_type=jnp.float32) + bh2_ref[...]
    logits = jax.lax.dot_general(feat, Wfc_ref[...], (((1,), (1,)), ((), ())),
                                 preferred_element_type=jnp.float32) + bfc_ref[...]
    lg_ref[...] = logits
    nrm = jnp.sqrt((feat_c * feat_c).sum(1, keepdims=True))
    nrm = jnp.maximum(nrm, 1e-12)
    fn_ref[...] = feat_c / nrm


def _final(h2raw, nA2, nD2, als2, ald2, b2, xres2, Wh1, bh1, Wh2, bh2,
           Wfcp, bfcp, R=256):
    grid = (NP // R,)
    return pl.pallas_call(
        _final_body,
        grid=grid,
        in_specs=[
            pl.BlockSpec((R, HID), lambda i: (i, 0)),
            pl.BlockSpec((R, HID), lambda i: (i, 0)),
            pl.BlockSpec((R, 16), lambda i: (i, 0)),
            pl.BlockSpec((R, 8), lambda i: (i, 0)),
            pl.BlockSpec((R, 8), lambda i: (i, 0)),
            pl.BlockSpec((1, HID), lambda i: (0, 0)),
            pl.BlockSpec((R, HID), lambda i: (i, 0)),
            pl.BlockSpec((HID, HID), lambda i: (0, 0)),
            pl.BlockSpec((1, HID), lambda i: (0, 0)),
            pl.BlockSpec((128, HID), lambda i: (0, 0)),
            pl.BlockSpec((1, 128), lambda i: (0, 0)),
            pl.BlockSpec((8, HID), lambda i: (0, 0)),
            pl.BlockSpec((1, 8), lambda i: (0, 0)),
        ],
        out_specs=[
            pl.BlockSpec((R, 8), lambda i: (i, 0)),
            pl.BlockSpec((R, 128), lambda i: (i, 0)),
        ],
        out_shape=[
            jax.ShapeDtypeStruct((NP, 8), jnp.float32),
            jax.ShapeDtypeStruct((NP, 128), jnp.float32),
        ],
        compiler_params=pltpu.CompilerParams(
            dimension_semantics=("arbitrary",)),
    )(h2raw, nA2, nD2, als2, ald2, b2, xres2, Wh1, bh1, Wh2, bh2, Wfcp, bfcp)


def _edge_jnp(feat, als, ald, src, dst, heads, width, n_valid):
    per = width // heads
    alpha = als[src, :heads] + ald[dst, :heads]
    alpha = jnp.where(alpha > 0, alpha, 0.2 * alpha)
    ee = jnp.exp(alpha)
    msg = (feat[src].reshape(-1, heads, per) * ee[:, :, None]).reshape(-1, width)
    nA = jax.ops.segment_sum(msg, dst, num_segments=NP)
    nDh = jax.ops.segment_sum(ee, dst, num_segments=NP)
    nD = jnp.pad(nDh, ((0, 0), (0, 16 - heads)))
    return nA, nD


def kernel(x, W1, att_src1, att_dst1, b1, W2, att_src2, att_dst2, b2,
           Wr1, br1, Wr2, br2, Wfc, bfc, Wh1, bh1, Wh2, bh2):
    n = x.shape[0]
    xp = jnp.pad(x, ((0, NP - n), (0, 0)))
    idx = _knn_topk(xp, n)
    src = jnp.repeat(jnp.arange(n), KNN)
    dst = idx[:n, :KNN].reshape(-1)

    as1p = jnp.pad(att_src1, ((0, 8 - HEADS), (0, 0)))
    ad1p = jnp.pad(att_dst1, ((0, 8 - HEADS), (0, 0)))
    as2p = jnp.pad(att_src2, ((0, 7), (0, 0)))
    ad2p = jnp.pad(att_dst2, ((0, 7), (0, 0)))

    h1, als1, ald1, xres1 = _proj1(xp, W1, Wr1, as1p, ad1p, br1[None, :])

    nA1, nD1 = _edge_jnp(h1, als1, ald1, src, dst, HEADS, HEADS * HID, n)

    h2raw, als2, ald2, xres2 = _mid(h1, nA1, nD1, als1, ald1, b1[None, :],
                                    xres1, W2, Wr2, as2p, ad2p, br2[None, :])

    nA2, nD2 = _edge_jnp(h2raw, als2, ald2, src, dst, 1, HID, n)

    Wfcp = jnp.pad(Wfc, ((0, 3), (0, 0)))
    bfcp = jnp.pad(bfc, (0, 3))[None, :]
    logits_p, featn_p = _final(h2raw, nA2, nD2, als2, ald2, b2[None, :],
                               xres2, Wh1, bh1[None, :], Wh2, bh2[None, :],
                               Wfcp, bfcp)
    return (logits_p[:n, :5], featn_p[:n])


# speedup vs baseline: 11.7265x; 1.4722x over previous
import functools
import jax
import jax.numpy as jnp
from jax import lax
from jax.experimental import pallas as pl
from jax.experimental.pallas import tpu as pltpu
from jax.experimental.pallas import tpu_sc as plsc

KNN = 5
HEADS = 4
HID = 768
NP = 10240


def _knn_body(xr_ref, xc_ref, o_ref, sim_ref, *, n_valid, R, C, NT, k):
    i = pl.program_id(0)
    j = pl.program_id(1)
    xr = xr_ref[...]
    xc = xc_ref[...]
    xr = xr * jax.lax.rsqrt(jnp.maximum((xr * xr).sum(1, keepdims=True), 1e-30))
    xc = xc * jax.lax.rsqrt(jnp.maximum((xc * xc).sum(1, keepdims=True), 1e-30))
    s = jax.lax.dot_general(xr, xc, (((1,), (1,)), ((), ())),
                            preferred_element_type=jnp.float32)
    row_g = i * R + jax.lax.broadcasted_iota(jnp.int32, (R, C), 0)
    col_g = j * C + jax.lax.broadcasted_iota(jnp.int32, (R, C), 1)
    s = jnp.where((col_g == row_g) | (col_g >= n_valid), -1e30, s)
    sim_ref[:, pl.ds(j * C, C)] = s

    @pl.when(j == pl.num_programs(1) - 1)
    def _():
        val = sim_ref[...]
        cols = jax.lax.broadcasted_iota(jnp.int32, (R, NT), 1)
        picks = []
        for _p in range(k):
            m = jnp.max(val, axis=1, keepdims=True)
            amin = jnp.min(jnp.where(val >= m, cols, NT), axis=1, keepdims=True)
            picks.append(amin)
            val = jnp.where(cols == amin, -jnp.float32(jnp.inf), val)
        picks += [jnp.zeros((R, 1), jnp.int32)] * (8 - k)
        o_ref[...] = jnp.concatenate(picks, axis=1)


def _knn_topk(xp, n_valid, k=KNN, R=256, C=2048):
    n_pad, D = xp.shape
    grid = (n_pad // R, n_pad // C)
    out = pl.pallas_call(
        functools.partial(_knn_body, n_valid=n_valid, R=R, C=C, NT=n_pad, k=k),
        grid=grid,
        in_specs=[
            pl.BlockSpec((R, D), lambda i, j: (i, 0)),
            pl.BlockSpec((C, D), lambda i, j: (j, 0)),
        ],
        out_specs=pl.BlockSpec((R, 8), lambda i, j: (i, 0)),
        out_shape=jax.ShapeDtypeStruct((n_pad, 8), jnp.int32),
        scratch_shapes=[pltpu.VMEM((R, n_pad), jnp.float32)],
        compiler_params=pltpu.CompilerParams(
            dimension_semantics=("parallel", "arbitrary")),
    )(xp, xp)
    return out


def _proj1_body(x_ref, W1_ref, Wr1_ref, as_ref, ad_ref, br1_ref,
                h1_ref, als_ref, ald_ref, xres_ref):
    x = x_ref[...]
    h1 = jax.lax.dot_general(x, W1_ref[...], (((1,), (0,)), ((), ())),
                             preferred_element_type=jnp.float32)
    h1_ref[...] = h1
    xres_ref[...] = jax.lax.dot_general(
        x, Wr1_ref[...], (((1,), (1,)), ((), ())),
        preferred_element_type=jnp.float32) + br1_ref[...]
    als, ald = [], []
    for h in range(HEADS):
        sl = h1[:, h * HID:(h + 1) * HID]
        als.append((sl * as_ref[h, :][None, :]).sum(1, keepdims=True))
        ald.append((sl * ad_ref[h, :][None, :]).sum(1, keepdims=True))
    zpad = [jnp.zeros_like(als[0])] * (8 - HEADS)
    als_ref[...] = jnp.concatenate(als + zpad, axis=1)
    ald_ref[...] = jnp.concatenate(ald + zpad, axis=1)


def _proj1(xp, W1, Wr1, asrc, adst, br1, R=256):
    D = xp.shape[1]
    DH = HEADS * HID
    grid = (NP // R,)
    return pl.pallas_call(
        _proj1_body,
        grid=grid,
        in_specs=[
            pl.BlockSpec((R, D), lambda i: (i, 0)),
            pl.BlockSpec((D, DH), lambda i: (0, 0)),
            pl.BlockSpec((DH, D), lambda i: (0, 0)),
            pl.BlockSpec((8, HID), lambda i: (0, 0)),
            pl.BlockSpec((8, HID), lambda i: (0, 0)),
            pl.BlockSpec((1, DH), lambda i: (0, 0)),
        ],
        out_specs=[
            pl.BlockSpec((R, DH), lambda i: (i, 0)),
            pl.BlockSpec((R, 8), lambda i: (i, 0)),
            pl.BlockSpec((R, 8), lambda i: (i, 0)),
            pl.BlockSpec((R, DH), lambda i: (i, 0)),
        ],
        out_shape=[
            jax.ShapeDtypeStruct((NP, DH), jnp.float32),
            jax.ShapeDtypeStruct((NP, 8), jnp.float32),
            jax.ShapeDtypeStruct((NP, 8), jnp.float32),
            jax.ShapeDtypeStruct((NP, DH), jnp.float32),
        ],
        compiler_params=pltpu.CompilerParams(
            dimension_semantics=("arbitrary",)),
    )(xp, W1, Wr1, asrc, adst, br1)


def _mid_body(h1_ref, nA_ref, nD_ref, als_ref, ald_ref, b1_ref, xres_ref,
              W2_ref, Wr2_ref, as2_ref, ad2_ref, br2_ref,
              h2_ref, als2_ref, ald2_ref, xres2_ref):
    als = als_ref[...]
    ald = ald_ref[...]
    a = als + ald
    aself = jnp.where(a > 0, a, 0.2 * a)
    es = jnp.exp(aself)
    denom = nD_ref[:, :8] + es + 1e-16
    h1 = h1_ref[...]
    nA = jnp.concatenate([nA_ref[p] for p in range(24)], axis=1)
    outs = []
    for h in range(HEADS):
        sl = slice(h * HID, (h + 1) * HID)
        agg = (nA[:, sl] + es[:, h:h + 1] * h1[:, sl]) / denom[:, h:h + 1]
        outs.append(agg)
    out1 = jnp.concatenate(outs, axis=1) + b1_ref[...]
    out1 = jnp.maximum(out1, 0.0) + xres_ref[...]
    h2 = jax.lax.dot_general(out1, W2_ref[...], (((1,), (0,)), ((), ())),
                             preferred_element_type=jnp.float32)
    h2_ref[...] = h2
    xres2_ref[...] = jax.lax.dot_general(
        out1, Wr2_ref[...], (((1,), (1,)), ((), ())),
        preferred_element_type=jnp.float32) + br2_ref[...]
    als2 = (h2 * as2_ref[0, :][None, :]).sum(1, keepdims=True)
    ald2 = (h2 * ad2_ref[0, :][None, :]).sum(1, keepdims=True)
    zpad = jnp.zeros((h2.shape[0], 7), jnp.float32)
    als2_ref[...] = jnp.concatenate([als2, zpad], axis=1)
    ald2_ref[...] = jnp.concatenate([ald2, zpad], axis=1)


def _mid(h1, nA1, nD1, als1, ald1, b1, xres1, W2, Wr2, as2, ad2, br2, R=256):
    DH = HEADS * HID
    grid = (NP // R,)
    return pl.pallas_call(
        _mid_body,
        grid=grid,
        in_specs=[
            pl.BlockSpec((R, DH), lambda i: (i, 0)),
            pl.BlockSpec((24, R, 128), lambda i: (0, i, 0)),
            pl.BlockSpec((R, 16), lambda i: (i, 0)),
            pl.BlockSpec((R, 8), lambda i: (i, 0)),
            pl.BlockSpec((R, 8), lambda i: (i, 0)),
            pl.BlockSpec((1, DH), lambda i: (0, 0)),
            pl.BlockSpec((R, DH), lambda i: (i, 0)),
            pl.BlockSpec((DH, HID), lambda i: (0, 0)),
            pl.BlockSpec((HID, DH), lambda i: (0, 0)),
            pl.BlockSpec((8, HID), lambda i: (0, 0)),
            pl.BlockSpec((8, HID), lambda i: (0, 0)),
            pl.BlockSpec((1, HID), lambda i: (0, 0)),
        ],
        out_specs=[
            pl.BlockSpec((R, HID), lambda i: (i, 0)),
            pl.BlockSpec((R, 8), lambda i: (i, 0)),
            pl.BlockSpec((R, 8), lambda i: (i, 0)),
            pl.BlockSpec((R, HID), lambda i: (i, 0)),
        ],
        out_shape=[
            jax.ShapeDtypeStruct((NP, HID), jnp.float32),
            jax.ShapeDtypeStruct((NP, 8), jnp.float32),
            jax.ShapeDtypeStruct((NP, 8), jnp.float32),
            jax.ShapeDtypeStruct((NP, HID), jnp.float32),
        ],
        compiler_params=pltpu.CompilerParams(
            dimension_semantics=("arbitrary",)),
    )(h1, nA1, nD1, als1, ald1, b1, xres1, W2, Wr2, as2, ad2, br2)


def _final_body(h2_ref, nA_ref, nD_ref, als_ref, ald_ref, b2_ref, xres_ref,
                Wh1_ref, bh1_ref, Wh2_ref, bh2_ref, Wfc_ref, bfc_ref,
                lg_ref, fn_ref):
    a = als_ref[:, 0:1] + ald_ref[:, 0:1]
    aself = jnp.where(a > 0, a, 0.2 * a)
    es = jnp.exp(aself)
    denom = nD_ref[:, 0:1] + es + 1e-16
    h2raw = h2_ref[...]
    nA = jnp.concatenate([nA_ref[p] for p in range(6)], axis=1)
    agg = (nA + es * h2raw) / denom + b2_ref[...]
    feat = jnp.maximum(agg, 0.0) + xres_ref[...]
    fc1 = jax.lax.dot_general(feat, Wh1_ref[...], (((1,), (1,)), ((), ())),
                              preferred_element_type=jnp.float32) + bh1_ref[...]
    fc1 = jnp.maximum(fc1, 0.0)
    feat_c = jax.lax.dot_general(fc1, Wh2_ref[...], (((1,), (1,)), ((), ())),
                                 preferred_element_type=jnp.float32) + bh2_ref[...]
    logits = jax.lax.dot_general(feat, Wfc_ref[...], (((1,), (1,)), ((), ())),
                                 preferred_element_type=jnp.float32) + bfc_ref[...]
    lg_ref[...] = logits
    nrm = jnp.sqrt((feat_c * feat_c).sum(1, keepdims=True))
    nrm = jnp.maximum(nrm, 1e-12)
    fn_ref[...] = feat_c / nrm


def _final(h2raw, nA2, nD2, als2, ald2, b2, xres2, Wh1, bh1, Wh2, bh2,
           Wfcp, bfcp, R=256):
    grid = (NP // R,)
    return pl.pallas_call(
        _final_body,
        grid=grid,
        in_specs=[
            pl.BlockSpec((R, HID), lambda i: (i, 0)),
            pl.BlockSpec((6, R, 128), lambda i: (0, i, 0)),
            pl.BlockSpec((R, 16), lambda i: (i, 0)),
            pl.BlockSpec((R, 8), lambda i: (i, 0)),
            pl.BlockSpec((R, 8), lambda i: (i, 0)),
            pl.BlockSpec((1, HID), lambda i: (0, 0)),
            pl.BlockSpec((R, HID), lambda i: (i, 0)),
            pl.BlockSpec((HID, HID), lambda i: (0, 0)),
            pl.BlockSpec((1, HID), lambda i: (0, 0)),
            pl.BlockSpec((128, HID), lambda i: (0, 0)),
            pl.BlockSpec((1, 128), lambda i: (0, 0)),
            pl.BlockSpec((8, HID), lambda i: (0, 0)),
            pl.BlockSpec((1, 8), lambda i: (0, 0)),
        ],
        out_specs=[
            pl.BlockSpec((R, 8), lambda i: (i, 0)),
            pl.BlockSpec((R, 128), lambda i: (i, 0)),
        ],
        out_shape=[
            jax.ShapeDtypeStruct((NP, 8), jnp.float32),
            jax.ShapeDtypeStruct((NP, 128), jnp.float32),
        ],
        compiler_params=pltpu.CompilerParams(
            dimension_semantics=("arbitrary",)),
    )(h2raw, nA2, nD2, als2, ald2, b2, xres2, Wh1, bh1, Wh2, bh2, Wfcp, bfcp)


def _edge_w(als16, ald16p, dst2, heads, n_e):
    E = dst2.shape[0]
    ept = E // 32
    nch = ept // 16
    AROWS = NP + 64
    mesh = plsc.VectorSubcoreMesh(core_axis_name="c", subcore_axis_name="s")

    @functools.partial(
        pl.kernel, mesh=mesh,
        compiler_params=pltpu.CompilerParams(use_tc_tiling_on_sc=False),
        out_type=[jax.ShapeDtypeStruct((E, 16), jnp.float32),
                  jax.ShapeDtypeStruct((2, NP, 16), jnp.float32)],
        scratch_types=[
            pltpu.VMEM((ept,), jnp.int32),
            pltpu.VMEM((16, 16), jnp.float32),
            pltpu.VMEM((16, 16), jnp.float32),
            pltpu.VMEM((16, 16), jnp.float32),
            pltpu.VMEM((4, 16), jnp.float32),
            pltpu.VMEM_SHARED((AROWS, 16), jnp.float32),
            pltpu.SemaphoreType.DMA,
            pltpu.SemaphoreType.DMA,
        ])
    def k(als_hbm, ald_hbm, dst_hbm, wE_hbm, nDp_hbm,
          dstv, albuf, adbuf, wbuf, zbuf, accD, sem1, sem2):
        c = lax.axis_index("c")
        s = lax.axis_index("s")
        e0 = c * (E // 2) + s * ept
        pltpu.sync_copy(dst_hbm.at[pl.ds(e0, ept)], dstv)

        def zb(i, _):
            zbuf[i, :] = jnp.zeros((16,), jnp.float32)
            return 0
        lax.fori_loop(0, 4, zb, 0)

        def zc(b, _):
            pltpu.sync_copy(zbuf, accD.at[pl.ds(s * (AROWS // 16) + b * 4, 4)])
            return 0
        lax.fori_loop(0, AROWS // 64, zc, 0)
        plsc.subcore_barrier()

        def chunk(kk, _):
            it = lax.iota(jnp.int32, 16)
            dvv = dstv[pl.ds(kk * 16, 16)]
            srcv = (e0 + kk * 16 + it) % NP
            cp1 = pltpu.async_copy(als_hbm.at[srcv], albuf, sem1)
            cp2 = pltpu.async_copy(ald_hbm.at[dvv], adbuf, sem2)
            cp1.wait()
            cp2.wait()
            mk = lax.shift_right_logical(it - heads, 31).astype(jnp.float32)
            for j in range(16):
                t = albuf[j, :] + adbuf[j, :]
                t = jnp.maximum(t, 0.2 * t)
                wbuf[j, :] = jnp.exp(t) * mk
            pltpu.sync_copy(wbuf, wE_hbm.at[pl.ds(e0 + kk * 16, 16)])
            pltpu.sync_copy(wbuf, accD.at[dvv], add=True)
            return 0
        lax.fori_loop(0, nch, chunk, 0)
        plsc.subcore_barrier()
        pltpu.sync_copy(accD.at[pl.ds(s * (NP // 16), NP // 16)],
                        nDp_hbm.at[c, pl.ds(s * (NP // 16), NP // 16)])

    return k(als16, ald16p, dst2)


def _wmsg_body(h1_ref, w_ref, o_ref):
    cc = pl.program_id(2)
    hb = h1_ref[:, pl.ds(pl.multiple_of(cc * 128, 128), 128)]
    wf = w_ref[...]
    lane = jax.lax.broadcasted_iota(jnp.int32, wf.shape, 1)
    w = jnp.sum(jnp.where(lane == cc // 6, wf, 0.0), axis=1, keepdims=True)
    o_ref[0] = hb * w


def _wmsg(h1, wE, W):
    E = wE.shape[0]
    CP = W // 128
    SB = 512
    grid = (NP // SB, KNN, CP)
    return pl.pallas_call(
        _wmsg_body,
        grid=grid,
        in_specs=[
            pl.BlockSpec((SB, W), lambda i, j, cc: (i, 0)),
            pl.BlockSpec((SB, 16), lambda i, j, cc: (j * (NP // SB) + i, 0)),
        ],
        out_specs=pl.BlockSpec((1, SB, 128),
                               lambda i, j, cc: (cc, j * (NP // SB) + i, 0)),
        out_shape=jax.ShapeDtypeStruct((CP, E, 128), jnp.float32),
        compiler_params=pltpu.CompilerParams(
            dimension_semantics=("parallel", "arbitrary", "arbitrary")),
    )(h1, wE)


def _edge_b(msg3, dst2d, CP):
    nch2, CH = dst2d.shape
    E = nch2 * CH
    ept = E // 16
    tch = ept // CH
    per_sc = CP // 2
    AROWS = NP + 64
    mesh = plsc.VectorSubcoreMesh(core_axis_name="c", subcore_axis_name="s")

    @functools.partial(
        pl.kernel, mesh=mesh,
        compiler_params=pltpu.CompilerParams(use_tc_tiling_on_sc=False),
        out_type=jax.ShapeDtypeStruct((CP, NP, 128), jnp.float32),
        scratch_types=[
            pltpu.VMEM((tch, CH), jnp.int32),
            pltpu.VMEM((CH, 128), jnp.float32),
            pltpu.VMEM((4, 128), jnp.float32),
            pltpu.VMEM_SHARED((AROWS, 128), jnp.float32),
            pltpu.SemaphoreType.DMA,
        ])
    def k(msg_hbm, dst_hbm, out_hbm, dstv2, rowbuf, zbuf, acc, sem):
        c = lax.axis_index("c")
        s = lax.axis_index("s")
        pltpu.sync_copy(dst_hbm.at[pl.ds(s * tch, tch)], dstv2)

        def zrow(i, _):
            def zl(q, _):
                zbuf[i, pl.ds(q * 16, 16)] = jnp.zeros((16,), jnp.float32)
                return 0
            lax.fori_loop(0, 8, zl, 0)
            return 0
        lax.fori_loop(0, 4, zrow, 0)

        for pp in range(per_sc):
            p = c * per_sc + pp

            def zc(b, _):
                pltpu.sync_copy(zbuf, acc.at[pl.ds(s * (AROWS // 16) + b * 4, 4)])
                return 0
            lax.fori_loop(0, AROWS // 64, zc, 0)
            plsc.subcore_barrier()

            def chunk(kk, _):
                cp = pltpu.async_copy(
                    msg_hbm.at[p, pl.ds(s * ept + kk * CH, CH)], rowbuf, sem)
                cp.wait()
                pltpu.sync_copy(rowbuf, acc.at[dstv2.at[kk]], add=True)
                return 0
            lax.fori_loop(0, tch, chunk, 0)
            plsc.subcore_barrier()
            pltpu.sync_copy(acc.at[pl.ds(s * (NP // 16), NP // 16)],
                            out_hbm.at[p, pl.ds(s * (NP // 16), NP // 16)])
            plsc.subcore_barrier()

    return k(msg3, dst2d)


def _edge_jnp(feat, als, ald, src, dst, heads, width, n_valid):
    per = width // heads
    alpha = als[src, :heads] + ald[dst, :heads]
    alpha = jnp.where(alpha > 0, alpha, 0.2 * alpha)
    ee = jnp.exp(alpha)
    msg = (feat[src].reshape(-1, heads, per) * ee[:, :, None]).reshape(-1, width)
    nA = jax.ops.segment_sum(msg, dst, num_segments=NP)
    nDh = jax.ops.segment_sum(ee, dst, num_segments=NP)
    nD = jnp.pad(nDh, ((0, 0), (0, 16 - heads)))
    return nA, nD


def kernel(x, W1, att_src1, att_dst1, b1, W2, att_src2, att_dst2, b2,
           Wr1, br1, Wr2, br2, Wfc, bfc, Wh1, bh1, Wh2, bh2):
    n = x.shape[0]
    xp = jnp.pad(x, ((0, NP - n), (0, 0)))
    idx = _knn_topk(xp, n)
    src = jnp.repeat(jnp.arange(n), KNN)
    dst = idx[:n, :KNN].reshape(-1)

    as1p = jnp.pad(att_src1, ((0, 8 - HEADS), (0, 0)))
    ad1p = jnp.pad(att_dst1, ((0, 8 - HEADS), (0, 0)))
    as2p = jnp.pad(att_src2, ((0, 7), (0, 0)))
    ad2p = jnp.pad(att_dst2, ((0, 7), (0, 0)))

    h1, als1, ald1, xres1 = _proj1(xp, W1, Wr1, as1p, ad1p, br1[None, :])

    valid = jnp.tile(jnp.arange(NP) < n, (KNN,))
    dst2 = jnp.where(valid, idx[:, :KNN].T.reshape(-1), NP).astype(jnp.int32)
    dst2d = dst2.reshape(-1, 64)

    als1_16 = jnp.pad(als1, ((0, 0), (0, 8)))
    ald1_16 = jnp.pad(ald1, ((0, 64), (0, 8)))
    wE1, nDp1 = _edge_w(als1_16, ald1_16, dst2, HEADS, n * KNN)
    nD1 = nDp1[0] + nDp1[1]
    msg1 = _wmsg(h1, wE1, HEADS * HID)
    nA1 = _edge_b(msg1, dst2d, 24)

    h2raw, als2, ald2, xres2 = _mid(h1, nA1, nD1, als1, ald1, b1[None, :],
                                    xres1, W2, Wr2, as2p, ad2p, br2[None, :])

    als2_16 = jnp.pad(als2, ((0, 0), (0, 8)))
    ald2_16 = jnp.pad(ald2, ((0, 64), (0, 8)))
    wE2, nDp2 = _edge_w(als2_16, ald2_16, dst2, 1, n * KNN)
    nD2 = nDp2[0] + nDp2[1]
    msg2 = _wmsg(h2raw, wE2, HID)
    nA2 = _edge_b(msg2, dst2d, 6)

    Wfcp = jnp.pad(Wfc, ((0, 3), (0, 0)))
    bfcp = jnp.pad(bfc, (0, 3))[None, :]
    logits_p, featn_p = _final(h2raw, nA2, nD2, als2, ald2, b2[None, :],
                               xres2, Wh1, bh1[None, :], Wh2, bh2[None, :],
                               Wfcp, bfcp)
    return (logits_p[:n, :5], featn_p[:n])


# speedup vs baseline: 12.9051x; 1.1005x over previous
import functools
import jax
import jax.numpy as jnp
from jax import lax
from jax.experimental import pallas as pl
from jax.experimental.pallas import tpu as pltpu
from jax.experimental.pallas import tpu_sc as plsc

KNN = 5
HEADS = 4
HID = 768
NP = 10240


def _knn_body(xr_ref, xc_ref, o_ref, sim_ref, *, n_valid, R, C, NT, k):
    i = pl.program_id(0)
    j = pl.program_id(1)
    xr = xr_ref[...]
    xc = xc_ref[...]
    xr = xr * jax.lax.rsqrt(jnp.maximum((xr * xr).sum(1, keepdims=True), 1e-30))
    xc = xc * jax.lax.rsqrt(jnp.maximum((xc * xc).sum(1, keepdims=True), 1e-30))
    s = jax.lax.dot_general(xr, xc, (((1,), (1,)), ((), ())),
                            preferred_element_type=jnp.float32)
    row_g = i * R + jax.lax.broadcasted_iota(jnp.int32, (R, C), 0)
    col_g = j * C + jax.lax.broadcasted_iota(jnp.int32, (R, C), 1)
    s = jnp.where((col_g == row_g) | (col_g >= n_valid), -1e30, s)
    sim_ref[:, pl.ds(j * C, C)] = s

    @pl.when(j == pl.num_programs(1) - 1)
    def _():
        val = sim_ref[...]
        cols = jax.lax.broadcasted_iota(jnp.int32, (R, NT), 1)
        picks = []
        for _p in range(k):
            m = jnp.max(val, axis=1, keepdims=True)
            amin = jnp.min(jnp.where(val >= m, cols, NT), axis=1, keepdims=True)
            picks.append(amin)
            val = jnp.where(cols == amin, -jnp.float32(jnp.inf), val)
        picks += [jnp.zeros((R, 1), jnp.int32)] * (8 - k)
        o_ref[...] = jnp.concatenate(picks, axis=1)


def _knn_topk(xp, n_valid, k=KNN, R=256, C=2048):
    n_pad, D = xp.shape
    grid = (n_pad // R, n_pad // C)
    out = pl.pallas_call(
        functools.partial(_knn_body, n_valid=n_valid, R=R, C=C, NT=n_pad, k=k),
        grid=grid,
        in_specs=[
            pl.BlockSpec((R, D), lambda i, j: (i, 0)),
            pl.BlockSpec((C, D), lambda i, j: (j, 0)),
        ],
        out_specs=pl.BlockSpec((R, 8), lambda i, j: (i, 0)),
        out_shape=jax.ShapeDtypeStruct((n_pad, 8), jnp.int32),
        scratch_shapes=[pltpu.VMEM((R, n_pad), jnp.float32)],
        compiler_params=pltpu.CompilerParams(
            dimension_semantics=("parallel", "arbitrary")),
    )(xp, xp)
    return out


def _proj1_body(x_ref, W1_ref, Wr1_ref, as_ref, ad_ref, br1_ref,
                h1_ref, als_ref, ald_ref, xres_ref):
    x = x_ref[...]
    h1 = jax.lax.dot_general(x, W1_ref[...], (((1,), (0,)), ((), ())),
                             preferred_element_type=jnp.float32)
    h1_ref[...] = h1
    xres_ref[...] = jax.lax.dot_general(
        x, Wr1_ref[...], (((1,), (1,)), ((), ())),
        preferred_element_type=jnp.float32) + br1_ref[...]
    als, ald = [], []
    for h in range(HEADS):
        sl = h1[:, h * HID:(h + 1) * HID]
        als.append((sl * as_ref[h, :][None, :]).sum(1, keepdims=True))
        ald.append((sl * ad_ref[h, :][None, :]).sum(1, keepdims=True))
    zpad = [jnp.zeros_like(als[0])] * (8 - HEADS)
    als_ref[...] = jnp.concatenate(als + zpad, axis=1)
    ald_ref[...] = jnp.concatenate(ald + zpad, axis=1)


def _proj1(xp, W1, Wr1, asrc, adst, br1, R=256):
    D = xp.shape[1]
    DH = HEADS * HID
    grid = (NP // R,)
    return pl.pallas_call(
        _proj1_body,
        grid=grid,
        in_specs=[
            pl.BlockSpec((R, D), lambda i: (i, 0)),
            pl.BlockSpec((D, DH), lambda i: (0, 0)),
            pl.BlockSpec((DH, D), lambda i: (0, 0)),
            pl.BlockSpec((8, HID), lambda i: (0, 0)),
            pl.BlockSpec((8, HID), lambda i: (0, 0)),
            pl.BlockSpec((1, DH), lambda i: (0, 0)),
        ],
        out_specs=[
            pl.BlockSpec((R, DH), lambda i: (i, 0)),
            pl.BlockSpec((R, 8), lambda i: (i, 0)),
            pl.BlockSpec((R, 8), lambda i: (i, 0)),
            pl.BlockSpec((R, DH), lambda i: (i, 0)),
        ],
        out_shape=[
            jax.ShapeDtypeStruct((NP, DH), jnp.float32),
            jax.ShapeDtypeStruct((NP, 8), jnp.float32),
            jax.ShapeDtypeStruct((NP, 8), jnp.float32),
            jax.ShapeDtypeStruct((NP, DH), jnp.float32),
        ],
        compiler_params=pltpu.CompilerParams(
            dimension_semantics=("arbitrary",)),
    )(xp, W1, Wr1, asrc, adst, br1)


def _mid_body(h1_ref, nA_ref, nD_ref, als_ref, ald_ref, b1_ref, xres_ref,
              W2_ref, Wr2_ref, as2_ref, ad2_ref, br2_ref,
              h2_ref, als2_ref, ald2_ref, xres2_ref):
    als = als_ref[...]
    ald = ald_ref[...]
    a = als + ald
    aself = jnp.where(a > 0, a, 0.2 * a)
    es = jnp.exp(aself)
    denom = nD_ref[:, :8] + es + 1e-16
    h1 = h1_ref[...]
    nA = jnp.concatenate([nA_ref[p] for p in range(24)], axis=1)
    outs = []
    for h in range(HEADS):
        sl = slice(h * HID, (h + 1) * HID)
        agg = (nA[:, sl] + es[:, h:h + 1] * h1[:, sl]) / denom[:, h:h + 1]
        outs.append(agg)
    out1 = jnp.concatenate(outs, axis=1) + b1_ref[...]
    out1 = jnp.maximum(out1, 0.0) + xres_ref[...]
    h2 = jax.lax.dot_general(out1, W2_ref[...], (((1,), (0,)), ((), ())),
                             preferred_element_type=jnp.float32)
    h2_ref[...] = h2
    xres2_ref[...] = jax.lax.dot_general(
        out1, Wr2_ref[...], (((1,), (1,)), ((), ())),
        preferred_element_type=jnp.float32) + br2_ref[...]
    als2 = (h2 * as2_ref[0, :][None, :]).sum(1, keepdims=True)
    ald2 = (h2 * ad2_ref[0, :][None, :]).sum(1, keepdims=True)
    zpad = jnp.zeros((h2.shape[0], 7), jnp.float32)
    als2_ref[...] = jnp.concatenate([als2, zpad], axis=1)
    ald2_ref[...] = jnp.concatenate([ald2, zpad], axis=1)


def _mid(h1, nA1, nD1, als1, ald1, b1, xres1, W2, Wr2, as2, ad2, br2, R=256):
    DH = HEADS * HID
    grid = (NP // R,)
    return pl.pallas_call(
        _mid_body,
        grid=grid,
        in_specs=[
            pl.BlockSpec((R, DH), lambda i: (i, 0)),
            pl.BlockSpec((24, R, 128), lambda i: (0, i, 0)),
            pl.BlockSpec((R, 16), lambda i: (i, 0)),
            pl.BlockSpec((R, 8), lambda i: (i, 0)),
            pl.BlockSpec((R, 8), lambda i: (i, 0)),
            pl.BlockSpec((1, DH), lambda i: (0, 0)),
            pl.BlockSpec((R, DH), lambda i: (i, 0)),
            pl.BlockSpec((DH, HID), lambda i: (0, 0)),
            pl.BlockSpec((HID, DH), lambda i: (0, 0)),
            pl.BlockSpec((8, HID), lambda i: (0, 0)),
            pl.BlockSpec((8, HID), lambda i: (0, 0)),
            pl.BlockSpec((1, HID), lambda i: (0, 0)),
        ],
        out_specs=[
            pl.BlockSpec((R, HID), lambda i: (i, 0)),
            pl.BlockSpec((R, 8), lambda i: (i, 0)),
            pl.BlockSpec((R, 8), lambda i: (i, 0)),
            pl.BlockSpec((R, HID), lambda i: (i, 0)),
        ],
        out_shape=[
            jax.ShapeDtypeStruct((NP, HID), jnp.float32),
            jax.ShapeDtypeStruct((NP, 8), jnp.float32),
            jax.ShapeDtypeStruct((NP, 8), jnp.float32),
            jax.ShapeDtypeStruct((NP, HID), jnp.float32),
        ],
        compiler_params=pltpu.CompilerParams(
            dimension_semantics=("arbitrary",)),
    )(h1, nA1, nD1, als1, ald1, b1, xres1, W2, Wr2, as2, ad2, br2)


def _final_body(h2_ref, nA_ref, nD_ref, als_ref, ald_ref, b2_ref, xres_ref,
                Wh1_ref, bh1_ref, Wh2_ref, bh2_ref, Wfc_ref, bfc_ref,
                lg_ref, fn_ref):
    a = als_ref[:, 0:1] + ald_ref[:, 0:1]
    aself = jnp.where(a > 0, a, 0.2 * a)
    es = jnp.exp(aself)
    denom = nD_ref[:, 0:1] + es + 1e-16
    h2raw = h2_ref[...]
    nA = jnp.concatenate([nA_ref[p] for p in range(6)], axis=1)
    agg = (nA + es * h2raw) / denom + b2_ref[...]
    feat = jnp.maximum(agg, 0.0) + xres_ref[...]
    fc1 = jax.lax.dot_general(feat, Wh1_ref[...], (((1,), (1,)), ((), ())),
                              preferred_element_type=jnp.float32) + bh1_ref[...]
    fc1 = jnp.maximum(fc1, 0.0)
    feat_c = jax.lax.dot_general(fc1, Wh2_ref[...], (((1,), (1,)), ((), ())),
                                 preferred_element_type=jnp.float32) + bh2_ref[...]
    logits = jax.lax.dot_general(feat, Wfc_ref[...], (((1,), (1,)), ((), ())),
                                 preferred_element_type=jnp.float32) + bfc_ref[...]
    lg_ref[...] = logits
    nrm = jnp.sqrt((feat_c * feat_c).sum(1, keepdims=True))
    nrm = jnp.maximum(nrm, 1e-12)
    fn_ref[...] = feat_c / nrm


def _final(h2raw, nA2, nD2, als2, ald2, b2, xres2, Wh1, bh1, Wh2, bh2,
           Wfcp, bfcp, R=256):
    grid = (NP // R,)
    return pl.pallas_call(
        _final_body,
        grid=grid,
        in_specs=[
            pl.BlockSpec((R, HID), lambda i: (i, 0)),
            pl.BlockSpec((6, R, 128), lambda i: (0, i, 0)),
            pl.BlockSpec((R, 16), lambda i: (i, 0)),
            pl.BlockSpec((R, 8), lambda i: (i, 0)),
            pl.BlockSpec((R, 8), lambda i: (i, 0)),
            pl.BlockSpec((1, HID), lambda i: (0, 0)),
            pl.BlockSpec((R, HID), lambda i: (i, 0)),
            pl.BlockSpec((HID, HID), lambda i: (0, 0)),
            pl.BlockSpec((1, HID), lambda i: (0, 0)),
            pl.BlockSpec((128, HID), lambda i: (0, 0)),
            pl.BlockSpec((1, 128), lambda i: (0, 0)),
            pl.BlockSpec((8, HID), lambda i: (0, 0)),
            pl.BlockSpec((1, 8), lambda i: (0, 0)),
        ],
        out_specs=[
            pl.BlockSpec((R, 8), lambda i: (i, 0)),
            pl.BlockSpec((R, 128), lambda i: (i, 0)),
        ],
        out_shape=[
            jax.ShapeDtypeStruct((NP, 8), jnp.float32),
            jax.ShapeDtypeStruct((NP, 128), jnp.float32),
        ],
        compiler_params=pltpu.CompilerParams(
            dimension_semantics=("arbitrary",)),
    )(h2raw, nA2, nD2, als2, ald2, b2, xres2, Wh1, bh1, Wh2, bh2, Wfcp, bfcp)


def _edge_w(als16, ald16p, dst2, heads, n_e):
    E = dst2.shape[0]
    ept = E // 32
    nch = ept // 16
    AROWS = NP + 64
    mesh = plsc.VectorSubcoreMesh(core_axis_name="c", subcore_axis_name="s")

    @functools.partial(
        pl.kernel, mesh=mesh,
        compiler_params=pltpu.CompilerParams(use_tc_tiling_on_sc=False),
        out_type=[jax.ShapeDtypeStruct((E, 16), jnp.float32),
                  jax.ShapeDtypeStruct((2, NP, 16), jnp.float32)],
        scratch_types=[
            pltpu.VMEM((ept,), jnp.int32),
            pltpu.VMEM((16, 16), jnp.float32),
            pltpu.VMEM((16, 16), jnp.float32),
            pltpu.VMEM((16, 16), jnp.float32),
            pltpu.VMEM((4, 16), jnp.float32),
            pltpu.VMEM_SHARED((AROWS, 16), jnp.float32),
            pltpu.SemaphoreType.DMA,
            pltpu.SemaphoreType.DMA,
        ])
    def k(als_hbm, ald_hbm, dst_hbm, wE_hbm, nDp_hbm,
          dstv, albuf, adbuf, wbuf, zbuf, accD, sem1, sem2):
        c = lax.axis_index("c")
        s = lax.axis_index("s")
        e0 = c * (E // 2) + s * ept
        pltpu.sync_copy(dst_hbm.at[pl.ds(e0, ept)], dstv)

        def zb(i, _):
            zbuf[i, :] = jnp.zeros((16,), jnp.float32)
            return 0
        lax.fori_loop(0, 4, zb, 0)

        def zc(b, _):
            pltpu.sync_copy(zbuf, accD.at[pl.ds(s * (AROWS // 16) + b * 4, 4)])
            return 0
        lax.fori_loop(0, AROWS // 64, zc, 0)
        plsc.subcore_barrier()

        def chunk(kk, _):
            it = lax.iota(jnp.int32, 16)
            dvv = dstv[pl.ds(kk * 16, 16)]
            srcv = (e0 + kk * 16 + it) % NP
            cp1 = pltpu.async_copy(als_hbm.at[srcv], albuf, sem1)
            cp2 = pltpu.async_copy(ald_hbm.at[dvv], adbuf, sem2)
            cp1.wait()
            cp2.wait()
            mk = lax.shift_right_logical(it - heads, 31).astype(jnp.float32)
            for j in range(16):
                t = albuf[j, :] + adbuf[j, :]
                t = jnp.maximum(t, 0.2 * t)
                wbuf[j, :] = jnp.exp(t) * mk
            pltpu.sync_copy(wbuf, wE_hbm.at[pl.ds(e0 + kk * 16, 16)])
            pltpu.sync_copy(wbuf, accD.at[dvv], add=True)
            return 0
        lax.fori_loop(0, nch, chunk, 0)
        plsc.subcore_barrier()
        pltpu.sync_copy(accD.at[pl.ds(s * (NP // 16), NP // 16)],
                        nDp_hbm.at[c, pl.ds(s * (NP // 16), NP // 16)])

    return k(als16, ald16p, dst2)


def _wmsg_body(h1_ref, w_ref, o_ref):
    cc = pl.program_id(2)
    hb = h1_ref[:, pl.ds(pl.multiple_of(cc * 128, 128), 128)]
    wf = w_ref[...]
    lane = jax.lax.broadcasted_iota(jnp.int32, wf.shape, 1)
    w = jnp.sum(jnp.where(lane == cc // 6, wf, 0.0), axis=1, keepdims=True)
    o_ref[0] = hb * w


def _wmsg(h1, wE, W):
    E = wE.shape[0]
    CP = W // 128
    SB = 512
    grid = (NP // SB, KNN, CP)
    return pl.pallas_call(
        _wmsg_body,
        grid=grid,
        in_specs=[
            pl.BlockSpec((SB, W), lambda i, j, cc: (i, 0)),
            pl.BlockSpec((SB, 16), lambda i, j, cc: (j * (NP // SB) + i, 0)),
        ],
        out_specs=pl.BlockSpec((1, SB, 128),
                               lambda i, j, cc: (cc, j * (NP // SB) + i, 0)),
        out_shape=jax.ShapeDtypeStruct((CP, E, 128), jnp.float32),
        compiler_params=pltpu.CompilerParams(
            dimension_semantics=("parallel", "arbitrary", "arbitrary")),
    )(h1, wE)


def _edge_b(msg3, dst2d, CP):
    nch2, CH = dst2d.shape
    E = nch2 * CH
    ept = E // 16
    tch = ept // CH
    per_sc = CP // 2
    AROWS = NP + 64
    mesh = plsc.VectorSubcoreMesh(core_axis_name="c", subcore_axis_name="s")

    @functools.partial(
        pl.kernel, mesh=mesh,
        compiler_params=pltpu.CompilerParams(use_tc_tiling_on_sc=False),
        out_type=jax.ShapeDtypeStruct((CP, NP, 128), jnp.float32),
        scratch_types=[
            pltpu.VMEM((tch, CH), jnp.int32),
            pltpu.VMEM((CH, 128), jnp.float32),
            pltpu.VMEM((CH, 128), jnp.float32),
            pltpu.VMEM((4, 128), jnp.float32),
            pltpu.VMEM_SHARED((AROWS, 128), jnp.float32),
            pltpu.SemaphoreType.DMA,
            pltpu.SemaphoreType.DMA,
        ])
    def k(msg_hbm, dst_hbm, out_hbm, dstv2, rowbuf, rowbuf2, zbuf, acc,
          sem, sem2):
        c = lax.axis_index("c")
        s = lax.axis_index("s")
        pltpu.sync_copy(dst_hbm.at[pl.ds(s * tch, tch)], dstv2)

        def zrow(i, _):
            def zl(q, _):
                zbuf[i, pl.ds(q * 16, 16)] = jnp.zeros((16,), jnp.float32)
                return 0
            lax.fori_loop(0, 8, zl, 0)
            return 0
        lax.fori_loop(0, 4, zrow, 0)

        for pp in range(per_sc):
            p = c * per_sc + pp

            def zc(b, _):
                pltpu.sync_copy(zbuf, acc.at[pl.ds(s * (AROWS // 16) + b * 4, 4)])
                return 0
            lax.fori_loop(0, AROWS // 64, zc, 0)
            plsc.subcore_barrier()

            def chunk(kk, _):
                cp = pltpu.async_copy(
                    msg_hbm.at[p, pl.ds(s * ept + kk * 2 * CH, CH)],
                    rowbuf, sem)
                cp2 = pltpu.async_copy(
                    msg_hbm.at[p, pl.ds(s * ept + (kk * 2 + 1) * CH, CH)],
                    rowbuf2, sem2)
                cp.wait()
                pltpu.sync_copy(rowbuf, acc.at[dstv2.at[kk * 2]], add=True)
                cp2.wait()
                pltpu.sync_copy(rowbuf2, acc.at[dstv2.at[kk * 2 + 1]],
                                add=True)
                return 0
            lax.fori_loop(0, tch // 2, chunk, 0)
            plsc.subcore_barrier()
            pltpu.sync_copy(acc.at[pl.ds(s * (NP // 16), NP // 16)],
                            out_hbm.at[p, pl.ds(s * (NP // 16), NP // 16)])
            plsc.subcore_barrier()

    return k(msg3, dst2d)


def _edge_jnp(feat, als, ald, src, dst, heads, width, n_valid):
    per = width // heads
    alpha = als[src, :heads] + ald[dst, :heads]
    alpha = jnp.where(alpha > 0, alpha, 0.2 * alpha)
    ee = jnp.exp(alpha)
    msg = (feat[src].reshape(-1, heads, per) * ee[:, :, None]).reshape(-1, width)
    nA = jax.ops.segment_sum(msg, dst, num_segments=NP)
    nDh = jax.ops.segment_sum(ee, dst, num_segments=NP)
    nD = jnp.pad(nDh, ((0, 0), (0, 16 - heads)))
    return nA, nD


def kernel(x, W1, att_src1, att_dst1, b1, W2, att_src2, att_dst2, b2,
           Wr1, br1, Wr2, br2, Wfc, bfc, Wh1, bh1, Wh2, bh2):
    n = x.shape[0]
    xp = jnp.pad(x, ((0, NP - n), (0, 0)))
    idx = _knn_topk(xp, n)
    src = jnp.repeat(jnp.arange(n), KNN)
    dst = idx[:n, :KNN].reshape(-1)

    as1p = jnp.pad(att_src1, ((0, 8 - HEADS), (0, 0)))
    ad1p = jnp.pad(att_dst1, ((0, 8 - HEADS), (0, 0)))
    as2p = jnp.pad(att_src2, ((0, 7), (0, 0)))
    ad2p = jnp.pad(att_dst2, ((0, 7), (0, 0)))

    h1, als1, ald1, xres1 = _proj1(xp, W1, Wr1, as1p, ad1p, br1[None, :])

    valid = jnp.tile(jnp.arange(NP) < n, (KNN,))
    dst2 = jnp.where(valid, idx[:, :KNN].T.reshape(-1), NP).astype(jnp.int32)
    dst2d = dst2.reshape(-1, 128)

    als1_16 = jnp.pad(als1, ((0, 0), (0, 8)))
    ald1_16 = jnp.pad(ald1, ((0, 64), (0, 8)))
    wE1, nDp1 = _edge_w(als1_16, ald1_16, dst2, HEADS, n * KNN)
    nD1 = nDp1[0] + nDp1[1]
    msg1 = _wmsg(h1, wE1, HEADS * HID)
    nA1 = _edge_b(msg1, dst2d, 24)

    h2raw, als2, ald2, xres2 = _mid(h1, nA1, nD1, als1, ald1, b1[None, :],
                                    xres1, W2, Wr2, as2p, ad2p, br2[None, :])

    als2_16 = jnp.pad(als2, ((0, 0), (0, 8)))
    ald2_16 = jnp.pad(ald2, ((0, 64), (0, 8)))
    wE2, nDp2 = _edge_w(als2_16, ald2_16, dst2, 1, n * KNN)
    nD2 = nDp2[0] + nDp2[1]
    msg2 = _wmsg(h2raw, wE2, HID)
    nA2 = _edge_b(msg2, dst2d, 6)

    Wfcp = jnp.pad(Wfc, ((0, 3), (0, 0)))
    bfcp = jnp.pad(bfc, (0, 3))[None, :]
    logits_p, featn_p = _final(h2raw, nA2, nD2, als2, ald2, b2[None, :],
                               xres2, Wh1, bh1[None, :], Wh2, bh2[None, :],
                               Wfcp, bfcp)
    return (logits_p[:n, :5], featn_p[:n])
